# baseline (speedup 1.0000x reference)
import functools

import jax
import jax.numpy as jnp
from jax import lax
from jax.experimental import pallas as pl
from jax.experimental.pallas import tpu as pltpu
from jax.experimental.pallas import tpu_sc as plsc

VOCAB = 1000000
DIM = 64
B = 4096
T = 200

NTOK = B * T
NC = 2
NS = 16
NW = NC * NS
PER_W = NTOK // NW
CHUNK = 128
NCHUNK = PER_W // CHUNK

_mesh = plsc.VectorSubcoreMesh(core_axis_name="c", subcore_axis_name="s")


@functools.partial(
    pl.kernel,
    out_type=jax.ShapeDtypeStruct((NTOK, DIM), jnp.float32),
    mesh=_mesh,
    scratch_types=[
        pltpu.VMEM((CHUNK,), jnp.int32),
        pltpu.VMEM((CHUNK, DIM), jnp.float32),
        pltpu.SemaphoreType.DMA,
    ],
    compiler_params=pltpu.CompilerParams(
        use_tc_tiling_on_sc=False, needs_layout_passes=False
    ),
)
def _emb_lookup(x_hbm, table_hbm, out_hbm, idx_v, rows_v, sem):
    wid = lax.axis_index("s") * NC + lax.axis_index("c")
    base = wid * PER_W

    def step(c, carry):
        off = base + c * CHUNK
        pltpu.sync_copy(x_hbm.at[pl.ds(off, CHUNK)], idx_v)
        pltpu.async_copy(table_hbm.at[idx_v], rows_v, sem).wait()

        zeros16 = jnp.zeros((16,), jnp.float32)
        for g in range(CHUNK // 16):
            iv = idx_v[pl.ds(g * 16, 16)]
            m = iv == 0
            npad = jnp.sum(m.astype(jnp.int32))

            @pl.when(npad > 0)
            def _fixup():
                rid = g * 16 + lax.iota(jnp.int32, 16)
                for k in range(DIM):
                    plsc.store_scatter(
                        rows_v,
                        [rid, jnp.full((16,), k, jnp.int32)],
                        zeros16,
                        mask=m,
                    )

        pltpu.sync_copy(rows_v, out_hbm.at[pl.ds(off, CHUNK)])
        return carry

    lax.fori_loop(0, NCHUNK, step, 0)


def kernel(x, table):
    xf = x.reshape(NTOK)
    out = _emb_lookup(xf, table)
    return out.reshape(B, T, DIM)


# speedup vs baseline: 1.4731x; 1.4731x over previous
import functools

import jax
import jax.numpy as jnp
from jax import lax
from jax.experimental import pallas as pl
from jax.experimental.pallas import tpu as pltpu
from jax.experimental.pallas import tpu_sc as plsc

VOCAB = 1000000
DIM = 64
B = 4096
T = 200

NTOK = B * T
NC = 2
NS = 16
NW = NC * NS
PER_W = NTOK // NW
CHUNK = 128
NCHUNK = PER_W // CHUNK
NBUF = 4
NSTEP = NCHUNK // NBUF - 1

_mesh = plsc.VectorSubcoreMesh(core_axis_name="c", subcore_axis_name="s")


@functools.partial(
    pl.kernel,
    out_type=jax.ShapeDtypeStruct((NTOK, 2 * DIM), jnp.float32),
    mesh=_mesh,
    scratch_types=(
        [pltpu.VMEM((NCHUNK, CHUNK), jnp.int32)]
        + [pltpu.VMEM((CHUNK, 2 * DIM), jnp.float32) for _ in range(NBUF)]
        + [pltpu.SemaphoreType.DMA for _ in range(2 * NBUF)]
    ),
    compiler_params=pltpu.CompilerParams(
        use_tc_tiling_on_sc=True, needs_layout_passes=False
    ),
)
def _emb_lookup(x_hbm, table_hbm, out_hbm, idx_all, *bufs_and_sems):
    rows = bufs_and_sems[:NBUF]
    sem_g = bufs_and_sems[NBUF : 2 * NBUF]
    sem_w = bufs_and_sems[2 * NBUF : 3 * NBUF]

    wid = lax.axis_index("s") * NC + lax.axis_index("c")
    base = wid * PER_W

    pltpu.sync_copy(x_hbm.at[pl.ds(wid * NCHUNK, NCHUNK)], idx_all)

    def gather(c, k):
        pltpu.async_copy(table_hbm.at[idx_all.at[c]], rows[k], sem_g[k])

    def wait_gather(k):
        pltpu.make_async_copy(out_hbm.at[pl.ds(0, CHUNK)], rows[k], sem_g[k]).wait()

    def write(c, k):
        pltpu.async_copy(
            rows[k], out_hbm.at[pl.ds(base + c * CHUNK, CHUNK)], sem_w[k]
        )

    def wait_write(k):
        pltpu.make_async_copy(rows[k], out_hbm.at[pl.ds(0, CHUNK)], sem_w[k]).wait()

    def fixup(c, k):
        zeros16 = jnp.zeros((16,), jnp.float32)
        for g in range(CHUNK // 16):
            iv = idx_all[c, pl.ds(g * 16, 16)]
            m = iv == 0
            npad = jnp.sum(m.astype(jnp.int32))

            @pl.when(npad > 0)
            def _():
                rid = g * 16 + lax.iota(jnp.int32, 16)
                for d in range(DIM):
                    plsc.store_scatter(
                        rows[k],
                        [rid, jnp.full((16,), d, jnp.int32)],
                        zeros16,
                        mask=m,
                    )

    for k in range(NBUF):
        gather(k, k)

    def step(j, carry):
        for k in range(NBUF):
            c = j * NBUF + k
            wait_gather(k)
            fixup(c, k)
            write(c, k)
            wait_write(k)
            gather(c + NBUF, k)
        return carry

    lax.fori_loop(0, NSTEP, step, 0)

    for k in range(NBUF):
        c = NSTEP * NBUF + k
        wait_gather(k)
        fixup(c, k)
        write(c, k)
    for k in range(NBUF):
        wait_write(k)


def kernel(x, table):
    emb = table.at[0].set(0.0)
    table_wide = jnp.pad(emb, ((0, 0), (0, DIM)))
    out = _emb_lookup(x.reshape(NTOK // CHUNK, CHUNK), table_wide)
    out64 = lax.slice(out, (0, 0), (NTOK, DIM))
    return out64.reshape(B, T, DIM)


# speedup vs baseline: 1.5673x; 1.0640x over previous
import functools

import jax
import jax.numpy as jnp
from jax import lax
from jax.experimental import pallas as pl
from jax.experimental.pallas import tpu as pltpu
from jax.experimental.pallas import tpu_sc as plsc

VOCAB = 1000000
DIM = 64
B = 4096
T = 200

NTOK = B * T
NC = 2
NS = 16
NW = NC * NS
PER_W = NTOK // NW
CHUNK = 128
NCHUNK = PER_W // CHUNK
NBUF = 4
NSTEP = NCHUNK // NBUF - 1

_mesh = plsc.VectorSubcoreMesh(core_axis_name="c", subcore_axis_name="s")

VB = 2048


def _widen_body(tt_ref, out_ref):
    t = tt_ref[...].T
    out_ref[...] = jnp.concatenate(
        [t, jnp.zeros((VB, DIM), jnp.float32)], axis=1
    )


_widen = pl.pallas_call(
    _widen_body,
    grid=(pl.cdiv(VOCAB, VB),),
    in_specs=[pl.BlockSpec((DIM, VB), lambda i: (0, i))],
    out_specs=pl.BlockSpec((VB, 2 * DIM), lambda i: (i, 0)),
    out_shape=jax.ShapeDtypeStruct((VOCAB, 2 * DIM), jnp.float32),
)


@functools.partial(
    pl.kernel,
    out_type=jax.ShapeDtypeStruct((NTOK, 2 * DIM), jnp.float32),
    mesh=_mesh,
    scratch_types=(
        [pltpu.VMEM((NCHUNK, CHUNK), jnp.int32)]
        + [pltpu.VMEM((CHUNK, 2 * DIM), jnp.float32) for _ in range(NBUF)]
        + [pltpu.SemaphoreType.DMA for _ in range(2 * NBUF)]
    ),
    compiler_params=pltpu.CompilerParams(
        use_tc_tiling_on_sc=True, needs_layout_passes=False
    ),
)
def _emb_lookup(x_hbm, table_hbm, out_hbm, idx_all, *bufs_and_sems):
    rows = bufs_and_sems[:NBUF]
    sem_g = bufs_and_sems[NBUF : 2 * NBUF]
    sem_w = bufs_and_sems[2 * NBUF : 3 * NBUF]

    wid = lax.axis_index("s") * NC + lax.axis_index("c")
    base = wid * PER_W

    pltpu.sync_copy(x_hbm.at[pl.ds(wid * NCHUNK, NCHUNK)], idx_all)

    def gather(c, k):
        pltpu.async_copy(table_hbm.at[idx_all.at[c]], rows[k], sem_g[k])

    def wait_gather(k):
        pltpu.make_async_copy(out_hbm.at[pl.ds(0, CHUNK)], rows[k], sem_g[k]).wait()

    def write(c, k):
        pltpu.async_copy(
            rows[k], out_hbm.at[pl.ds(base + c * CHUNK, CHUNK)], sem_w[k]
        )

    def wait_write(k):
        pltpu.make_async_copy(rows[k], out_hbm.at[pl.ds(0, CHUNK)], sem_w[k]).wait()

    def fixup(c, k):
        zeros16 = jnp.zeros((16,), jnp.float32)
        for g in range(CHUNK // 16):
            iv = idx_all[c, pl.ds(g * 16, 16)]
            m = iv == 0
            npad = jnp.sum(m.astype(jnp.int32))

            @pl.when(npad > 0)
            def _():
                rid = g * 16 + lax.iota(jnp.int32, 16)
                for d in range(DIM):
                    plsc.store_scatter(
                        rows[k],
                        [rid, jnp.full((16,), d, jnp.int32)],
                        zeros16,
                        mask=m,
                    )

    for k in range(NBUF):
        gather(k, k)

    def step(j, carry):
        for k in range(NBUF):
            c = j * NBUF + k
            wait_gather(k)
            fixup(c, k)
            write(c, k)
            wait_write(k)
            gather(c + NBUF, k)
        return carry

    lax.fori_loop(0, NSTEP, step, 0)

    for k in range(NBUF):
        c = NSTEP * NBUF + k
        wait_gather(k)
        fixup(c, k)
        write(c, k)
    for k in range(NBUF):
        wait_write(k)


def kernel(x, table):
    table_wide = _widen(table.T)
    out = _emb_lookup(x.reshape(NTOK // CHUNK, CHUNK), table_wide)
    out64 = lax.slice(out, (0, 0), (NTOK, DIM))
    return out64.reshape(B, T, DIM)
